# baseline (speedup 1.0000x reference)
import jax
import jax.numpy as jnp
from jax import lax
from jax.experimental import pallas as pl
from jax.experimental.pallas import tpu as pltpu
from jax.experimental.pallas import tpu_sc as plsc

NUM_ENTITIES = 1000000
NUM_RELATIONS = 1000
DIM = 64
BATCH = 16384
MARGIN = 1.0

_INFO = plsc.get_sparse_core_info()
NUM_CORES = _INFO.num_cores
NUM_SUBCORES = _INFO.num_subcores
NUM_WORKERS = NUM_CORES * NUM_SUBCORES
LANES = _INFO.num_lanes

PER_WORKER = BATCH // NUM_WORKERS
CHUNK = 128
NCHUNK = PER_WORKER // CHUNK
GROUPS = CHUNK // LANES
WIDE = 2 * DIM


def _rsqrt(x):
    i = plsc.bitcast(x, jnp.int32)
    i = jnp.int32(0x5F3759DF) - lax.shift_right_logical(i, 1)
    y = plsc.bitcast(i, jnp.float32)
    for _ in range(3):
        y = y * (jnp.float32(1.5) - jnp.float32(0.5) * x * y * y)
    return y


def _dist(sq):
    x = sq + jnp.float32(1e-12)
    return x * _rsqrt(x)


def _sc_kernel(h_row, r_row, t_row, h_colb, r_colb, t_colb, ent, rel, out,
               hi_v, ri_v, ti_v, hc_v, rc_v, tc_v,
               hp_v, rp_v, tp_v, hn_v, rn_v, tn_v,
               out_v, sems):
    wid = lax.axis_index("s") * NUM_CORES + lax.axis_index("c")
    base = wid * PER_WORKER

    for src, dst in ((h_row, hi_v), (r_row, ri_v), (t_row, ti_v),
                     (h_colb, hc_v), (r_colb, rc_v), (t_colb, tc_v)):
        pltpu.sync_copy(src.at[pl.ds(base, PER_WORKER)], dst.at[0])
        pltpu.sync_copy(src.at[pl.ds(BATCH + base, PER_WORKER)], dst.at[1])

    def chunk_body(c, _):
        co = c * CHUNK
        cps = [
            pltpu.async_copy(ent.at[hi_v.at[0, pl.ds(co, CHUNK)]], hp_v, sems.at[0]),
            pltpu.async_copy(rel.at[ri_v.at[0, pl.ds(co, CHUNK)]], rp_v, sems.at[1]),
            pltpu.async_copy(ent.at[ti_v.at[0, pl.ds(co, CHUNK)]], tp_v, sems.at[2]),
            pltpu.async_copy(ent.at[hi_v.at[1, pl.ds(co, CHUNK)]], hn_v, sems.at[3]),
            pltpu.async_copy(rel.at[ri_v.at[1, pl.ds(co, CHUNK)]], rn_v, sems.at[4]),
            pltpu.async_copy(ent.at[ti_v.at[1, pl.ds(co, CHUNK)]], tn_v, sems.at[5]),
        ]
        for cp in cps:
            cp.wait()

        def group_body(g, _):
            rows = lax.iota(jnp.int32, LANES) + g * LANES
            go = co + g * LANES

            def sq_sum(p, h_v, r_v, t_v):
                hc = hc_v[p, pl.ds(go, LANES)]
                rc = rc_v[p, pl.ds(go, LANES)]
                tc = tc_v[p, pl.ds(go, LANES)]
                acc = jnp.zeros((LANES,), jnp.float32)
                for d in range(DIM):
                    dd = jnp.full((LANES,), d, jnp.int32)
                    hv = plsc.load_gather(h_v, [rows, hc + dd])
                    rv = plsc.load_gather(r_v, [rows, rc + dd])
                    tv = plsc.load_gather(t_v, [rows, tc + dd])
                    df = hv + rv - tv
                    acc = acc + df * df
                return acc

            psq = sq_sum(0, hp_v, rp_v, tp_v)
            nsq = sq_sum(1, hn_v, rn_v, tn_v)
            score = jnp.maximum(jnp.float32(MARGIN) + _dist(psq) - _dist(nsq),
                                jnp.float32(0.0))
            out_v[pl.ds(g * LANES, LANES)] = score
            return 0

        lax.fori_loop(0, GROUPS, group_body, 0)
        pltpu.sync_copy(out_v, out.at[pl.ds(base + co, CHUNK)])
        return 0

    lax.fori_loop(0, NCHUNK, chunk_body, 0)


@jax.jit
def _transe_scores(h_row, r_row, t_row, h_colb, r_colb, t_colb, ent, rel):
    mesh = plsc.VectorSubcoreMesh(core_axis_name="c", subcore_axis_name="s")
    run = pl.kernel(
        _sc_kernel,
        out_type=jax.ShapeDtypeStruct((BATCH,), jnp.float32),
        mesh=mesh,
        scratch_types=[
            pltpu.VMEM((2, PER_WORKER), jnp.int32),
            pltpu.VMEM((2, PER_WORKER), jnp.int32),
            pltpu.VMEM((2, PER_WORKER), jnp.int32),
            pltpu.VMEM((2, PER_WORKER), jnp.int32),
            pltpu.VMEM((2, PER_WORKER), jnp.int32),
            pltpu.VMEM((2, PER_WORKER), jnp.int32),
            pltpu.VMEM((CHUNK, WIDE), jnp.float32),
            pltpu.VMEM((CHUNK, WIDE), jnp.float32),
            pltpu.VMEM((CHUNK, WIDE), jnp.float32),
            pltpu.VMEM((CHUNK, WIDE), jnp.float32),
            pltpu.VMEM((CHUNK, WIDE), jnp.float32),
            pltpu.VMEM((CHUNK, WIDE), jnp.float32),
            pltpu.VMEM((CHUNK,), jnp.float32),
            pltpu.SemaphoreType.DMA((6,)),
        ],
        compiler_params=pltpu.CompilerParams(
            needs_layout_passes=False, use_tc_tiling_on_sc=False),
    )
    return run(h_row, r_row, t_row, h_colb, r_colb, t_colb, ent, rel)


def kernel(batch, corrupted_batch, entity_emb, relation_emb):
    h_idx = jnp.concatenate([batch[:, 0], corrupted_batch[:, 0]])
    r_idx = jnp.concatenate([batch[:, 1], corrupted_batch[:, 1]]) % NUM_RELATIONS
    t_idx = jnp.concatenate([batch[:, 2], corrupted_batch[:, 2]])
    ent = entity_emb.reshape(NUM_ENTITIES // 2, WIDE)
    rel = relation_emb.reshape(NUM_RELATIONS // 2, WIDE)
    return _transe_scores(
        h_idx >> 1, r_idx >> 1, t_idx >> 1,
        (h_idx & 1) * DIM, (r_idx & 1) * DIM, (t_idx & 1) * DIM,
        ent, rel)


# speedup vs baseline: 1.0149x; 1.0149x over previous
import jax
import jax.numpy as jnp
from jax import lax
from jax.experimental import pallas as pl
from jax.experimental.pallas import tpu as pltpu
from jax.experimental.pallas import tpu_sc as plsc

NUM_ENTITIES = 1000000
NUM_RELATIONS = 1000
DIM = 64
BATCH = 16384
MARGIN = 1.0

_INFO = plsc.get_sparse_core_info()
NUM_CORES = _INFO.num_cores
NUM_SUBCORES = _INFO.num_subcores
NUM_WORKERS = NUM_CORES * NUM_SUBCORES
LANES = _INFO.num_lanes

PER_WORKER = BATCH // NUM_WORKERS
CHUNK = 64
NCHUNK = PER_WORKER // CHUNK
GROUPS = CHUNK // LANES
WIDE = 2 * DIM


def _rsqrt(x):
    i = plsc.bitcast(x, jnp.int32)
    i = jnp.int32(0x5F3759DF) - lax.shift_right_logical(i, 1)
    y = plsc.bitcast(i, jnp.float32)
    for _ in range(3):
        y = y * (jnp.float32(1.5) - jnp.float32(0.5) * x * y * y)
    return y


def _dist(sq):
    x = sq + jnp.float32(1e-12)
    return x * _rsqrt(x)


def _sc_kernel(h_row, r_row, t_row, h_colb, r_colb, t_colb, ent, rel, out,
               hi_p, ri_p, ti_p, hi_n, ri_n, ti_n,
               hc_p, rc_p, tc_p, hc_n, rc_n, tc_n,
               hp_v, rp_v, tp_v, hn_v, rn_v, tn_v,
               out_v, sems):
    wid = lax.axis_index("s") * NUM_CORES + lax.axis_index("c")
    base = wid * PER_WORKER

    idx_cps = []
    for i, (src, dst_p, dst_n) in enumerate(
            ((h_row, hi_p, hi_n), (r_row, ri_p, ri_n), (t_row, ti_p, ti_n),
             (h_colb, hc_p, hc_n), (r_colb, rc_p, rc_n),
             (t_colb, tc_p, tc_n))):
        idx_cps.append(pltpu.async_copy(
            src.at[pl.ds(base, PER_WORKER)], dst_p, sems.at[0, i]))
        idx_cps.append(pltpu.async_copy(
            src.at[pl.ds(BATCH + base, PER_WORKER)], dst_n, sems.at[1, i]))
    for cp in idx_cps:
        cp.wait()

    def fire(c, par):
        co = c * CHUNK
        pltpu.async_copy(ent.at[hi_p.at[pl.ds(co, CHUNK)]], hp_v.at[par],
                         sems.at[par, 0])
        pltpu.async_copy(rel.at[ri_p.at[pl.ds(co, CHUNK)]], rp_v.at[par],
                         sems.at[par, 1])
        pltpu.async_copy(ent.at[ti_p.at[pl.ds(co, CHUNK)]], tp_v.at[par],
                         sems.at[par, 2])
        pltpu.async_copy(ent.at[hi_n.at[pl.ds(co, CHUNK)]], hn_v.at[par],
                         sems.at[par, 3])
        pltpu.async_copy(rel.at[ri_n.at[pl.ds(co, CHUNK)]], rn_v.at[par],
                         sems.at[par, 4])
        pltpu.async_copy(ent.at[ti_n.at[pl.ds(co, CHUNK)]], tn_v.at[par],
                         sems.at[par, 5])

    def drain(par):
        pltpu.make_async_copy(ent.at[hi_p.at[pl.ds(0, CHUNK)]], hp_v.at[par],
                              sems.at[par, 0]).wait()
        pltpu.make_async_copy(rel.at[ri_p.at[pl.ds(0, CHUNK)]], rp_v.at[par],
                              sems.at[par, 1]).wait()
        pltpu.make_async_copy(ent.at[ti_p.at[pl.ds(0, CHUNK)]], tp_v.at[par],
                              sems.at[par, 2]).wait()
        pltpu.make_async_copy(ent.at[hi_n.at[pl.ds(0, CHUNK)]], hn_v.at[par],
                              sems.at[par, 3]).wait()
        pltpu.make_async_copy(rel.at[ri_n.at[pl.ds(0, CHUNK)]], rn_v.at[par],
                              sems.at[par, 4]).wait()
        pltpu.make_async_copy(ent.at[ti_n.at[pl.ds(0, CHUNK)]], tn_v.at[par],
                              sems.at[par, 5]).wait()

    fire(0, 0)

    def chunk_body(c, _):
        par = lax.rem(c, 2)
        co = c * CHUNK

        @pl.when(c + 1 < NCHUNK)
        def _prefetch():
            @pl.when(par == 0)
            def _():
                fire(c + 1, 1)

            @pl.when(par == 1)
            def _():
                fire(c + 1, 0)

        @pl.when(par == 0)
        def _():
            drain(0)

        @pl.when(par == 1)
        def _():
            drain(1)

        pv = jnp.full((LANES,), par, jnp.int32)

        def group_body(g, _):
            rows = lax.iota(jnp.int32, LANES) + g * LANES
            go = co + g * LANES
            hcp = hc_p[pl.ds(go, LANES)]
            rcp = rc_p[pl.ds(go, LANES)]
            tcp = tc_p[pl.ds(go, LANES)]
            hcn = hc_n[pl.ds(go, LANES)]
            rcn = rc_n[pl.ds(go, LANES)]
            tcn = tc_n[pl.ds(go, LANES)]
            one = jnp.full((LANES,), 1, jnp.int32)
            accp = jnp.zeros((LANES,), jnp.float32)
            accn = jnp.zeros((LANES,), jnp.float32)
            for _d in range(DIM):
                hv = plsc.load_gather(hp_v, [pv, rows, hcp])
                rv = plsc.load_gather(rp_v, [pv, rows, rcp])
                tv = plsc.load_gather(tp_v, [pv, rows, tcp])
                hw = plsc.load_gather(hn_v, [pv, rows, hcn])
                rw = plsc.load_gather(rn_v, [pv, rows, rcn])
                tw = plsc.load_gather(tn_v, [pv, rows, tcn])
                dfp = hv + rv - tv
                dfn = hw + rw - tw
                accp = accp + dfp * dfp
                accn = accn + dfn * dfn
                hcp = hcp + one
                rcp = rcp + one
                tcp = tcp + one
                hcn = hcn + one
                rcn = rcn + one
                tcn = tcn + one
            score = jnp.maximum(
                jnp.float32(MARGIN) + _dist(accp) - _dist(accn),
                jnp.float32(0.0))
            out_v[pl.ds(go, LANES)] = score
            return 0

        lax.fori_loop(0, GROUPS, group_body, 0)
        return 0

    lax.fori_loop(0, NCHUNK, chunk_body, 0)
    pltpu.sync_copy(out_v, out.at[pl.ds(base, PER_WORKER)])


@jax.jit
def _transe_scores(h_row, r_row, t_row, h_colb, r_colb, t_colb, ent, rel):
    mesh = plsc.VectorSubcoreMesh(core_axis_name="c", subcore_axis_name="s")
    run = pl.kernel(
        _sc_kernel,
        out_type=jax.ShapeDtypeStruct((BATCH,), jnp.float32),
        mesh=mesh,
        scratch_types=(
            [pltpu.VMEM((PER_WORKER,), jnp.int32) for _ in range(12)]
            + [pltpu.VMEM((2, CHUNK, WIDE), jnp.float32) for _ in range(6)]
            + [pltpu.VMEM((PER_WORKER,), jnp.float32),
               pltpu.SemaphoreType.DMA((2, 6))]
        ),
        compiler_params=pltpu.CompilerParams(
            needs_layout_passes=False, use_tc_tiling_on_sc=True),
    )
    return run(h_row, r_row, t_row, h_colb, r_colb, t_colb, ent, rel)


def kernel(batch, corrupted_batch, entity_emb, relation_emb):
    h_idx = jnp.concatenate([batch[:, 0], corrupted_batch[:, 0]])
    r_idx = jnp.concatenate([batch[:, 1], corrupted_batch[:, 1]]) % NUM_RELATIONS
    t_idx = jnp.concatenate([batch[:, 2], corrupted_batch[:, 2]])
    ent = entity_emb.reshape(NUM_ENTITIES // 2, WIDE)
    rel = relation_emb.reshape(NUM_RELATIONS // 2, WIDE)
    return _transe_scores(
        h_idx >> 1, r_idx >> 1, t_idx >> 1,
        (h_idx & 1) * DIM, (r_idx & 1) * DIM, (t_idx & 1) * DIM,
        ent, rel)
